# baseline (speedup 1.0000x reference)
import functools

import jax
import jax.numpy as jnp
from jax import lax
from jax.experimental import pallas as pl
from jax.experimental.pallas import tpu as pltpu
from jax.experimental.pallas import tpu_sc as plsc

D_MODEL = 64
LANES = 128
SUB = 8

_info = plsc.get_sparse_core_info()
_NC, _NS = _info.num_cores, _info.num_subcores
NW = _NC * _NS


def _make_sc_lookup(batch: int, seq: int, n_var: int):
    n_it = batch // LANES
    it_w = n_it // NW
    half = it_w // 2
    nlk = half * LANES
    lk_w = it_w * LANES
    mesh = plsc.VectorSubcoreMesh(core_axis_name="c", subcore_axis_name="s")

    @functools.partial(
        pl.kernel,
        mesh=mesh,
        out_type=jax.ShapeDtypeStruct(
            (seq, D_MODEL // SUB, n_it, SUB, LANES), jnp.float32
        ),
        scratch_types=[
            pltpu.VMEM((n_var * (D_MODEL + 1),), jnp.float32),
            pltpu.VMEM((lk_w,), jnp.int32),
            pltpu.VMEM((lk_w,), jnp.int32),
            pltpu.VMEM((D_MODEL // SUB, half, SUB, LANES), jnp.float32),
            pltpu.VMEM((D_MODEL // SUB, half, SUB, LANES), jnp.float32),
            pltpu.SemaphoreType.DMA,
            pltpu.SemaphoreType.DMA,
            pltpu.SemaphoreType.DMA,
            pltpu.SemaphoreType.DMA,
        ],
        compiler_params=pltpu.CompilerParams(
            use_tc_tiling_on_sc=False, needs_layout_passes=False
        ),
    )
    def sc_lookup(xt_hbm, tab_hbm, z_hbm, tab_v, idx_a, idx_b, buf_a, buf_b,
                  isem_a, isem_b, wsem_a, wsem_b):
        w = lax.axis_index("s") * _NC + lax.axis_index("c")
        col0 = w * lk_w
        pltpu.sync_copy(tab_hbm, tab_v)
        pltpu.async_copy(xt_hbm.at[0, pl.ds(col0, lk_w)], idx_a, isem_a)
        pltpu.async_copy(xt_hbm.at[1, pl.ds(col0, lk_w)], idx_b, isem_b)

        def compute_half(idx_v, h, buf):
            @plsc.parallel_loop(0, nlk // 16, unroll=6)
            def ilg_body(ilg):
                v = idx_v[pl.ds(h * nlk + ilg * 16, 16)]
                base = v * (D_MODEL + 1)
                itl = ilg // SUB
                il0 = (ilg % SUB) * 16
                for d in range(D_MODEL):
                    vec = plsc.load_gather(tab_v, [base + d])
                    buf[d // SUB, itl, d % SUB, pl.ds(il0, 16)] = vec

        def phase(s, idx_v, isem):
            pltpu.make_async_copy(
                xt_hbm.at[0, pl.ds(col0, lk_w)], idx_v, isem
            ).wait()
            for h, buf, wsem in ((0, buf_a, wsem_a), (1, buf_b, wsem_b)):
                dst = z_hbm.at[s, :, pl.ds(it_w * w + h * half, half)]

                @pl.when(s > 0)
                def _():
                    pltpu.make_async_copy(buf, dst, wsem).wait()

                compute_half(idx_v, h, buf)
                pltpu.async_copy(buf, dst, wsem)

            @pl.when(s + 2 < seq)
            def _():
                pltpu.async_copy(
                    xt_hbm.at[s + 2, pl.ds(col0, lk_w)], idx_v, isem
                )

        def body(p, carry):
            phase(2 * p, idx_a, isem_a)
            phase(2 * p + 1, idx_b, isem_b)
            return carry

        lax.fori_loop(0, seq // 2, body, 0)
        pltpu.make_async_copy(
            buf_a, z_hbm.at[seq - 1, :, pl.ds(it_w * w, half)], wsem_a
        ).wait()
        pltpu.make_async_copy(
            buf_b, z_hbm.at[seq - 1, :, pl.ds(it_w * w + half, half)], wsem_b
        ).wait()

    return sc_lookup


def kernel(x, emb_table):
    batch, seq = x.shape
    n_var, d_model = emb_table.shape
    assert d_model == D_MODEL and batch % (LANES * NW * 2) == 0 and seq % 2 == 0
    xt = x.astype(jnp.int32).T
    tab = jnp.pad(emb_table, ((0, 0), (0, 1))).reshape(-1)
    z = _make_sc_lookup(batch, seq, n_var)(xt, tab)
    return z.transpose(2, 4, 0, 1, 3).reshape(batch, seq, D_MODEL)


# speedup vs baseline: 1.9270x; 1.9270x over previous
import functools

import jax
import jax.numpy as jnp
from jax import lax
from jax.experimental import pallas as pl
from jax.experimental.pallas import tpu as pltpu
from jax.experimental.pallas import tpu_sc as plsc

D_MODEL = 64
LANES = 128
SUB = 8

_info = plsc.get_sparse_core_info()
_NC, _NS = _info.num_cores, _info.num_subcores
NW = _NC * _NS


def _make_sc_lookup(batch: int, seq: int, n_var: int):
    n_it = batch // LANES
    it_w = n_it // NW
    half = it_w // 2
    nlk = half * LANES
    lk_w = it_w * LANES
    mesh = plsc.VectorSubcoreMesh(core_axis_name="c", subcore_axis_name="s")

    @functools.partial(
        pl.kernel,
        mesh=mesh,
        out_type=jax.ShapeDtypeStruct(
            (seq, D_MODEL // SUB, n_it, SUB, LANES), jnp.float32
        ),
        scratch_types=[
            pltpu.VMEM((n_var * (D_MODEL + 1),), jnp.float32),
            pltpu.VMEM((lk_w,), jnp.int32),
            pltpu.VMEM((lk_w,), jnp.int32),
            pltpu.VMEM((D_MODEL // SUB, half, SUB, LANES), jnp.float32),
            pltpu.VMEM((D_MODEL // SUB, half, SUB, LANES), jnp.float32),
            pltpu.SemaphoreType.DMA,
            pltpu.SemaphoreType.DMA,
            pltpu.SemaphoreType.DMA,
            pltpu.SemaphoreType.DMA,
        ],
        compiler_params=pltpu.CompilerParams(
            use_tc_tiling_on_sc=False, needs_layout_passes=False
        ),
    )
    def sc_lookup(xt_hbm, tab_hbm, z_hbm, tab_v, idx_a, idx_b, buf_a, buf_b,
                  isem_a, isem_b, wsem_a, wsem_b):
        w = lax.axis_index("s") * _NC + lax.axis_index("c")
        col0 = w * lk_w
        pltpu.sync_copy(tab_hbm, tab_v)
        pltpu.async_copy(xt_hbm.at[0, pl.ds(col0, lk_w)], idx_a, isem_a)
        pltpu.async_copy(xt_hbm.at[1, pl.ds(col0, lk_w)], idx_b, isem_b)

        def compute_half(idx_v, h, buf):
            @plsc.parallel_loop(0, nlk // 16, unroll=4)
            def ilg_body(ilg):
                v = idx_v[pl.ds(h * nlk + ilg * 16, 16)]
                base = v * (D_MODEL + 1)
                itl = ilg // SUB
                il0 = (ilg % SUB) * 16
                for d in range(D_MODEL):
                    vec = plsc.load_gather(tab_v, [base + d])
                    buf[d // SUB, itl, d % SUB, pl.ds(il0, 16)] = vec

        def phase(s, idx_v, isem):
            pltpu.make_async_copy(
                xt_hbm.at[0, pl.ds(col0, lk_w)], idx_v, isem
            ).wait()
            for h, buf, wsem in ((0, buf_a, wsem_a), (1, buf_b, wsem_b)):
                dst = z_hbm.at[s, :, pl.ds(it_w * w + h * half, half)]

                @pl.when(s > 0)
                def _():
                    pltpu.make_async_copy(buf, dst, wsem).wait()

                compute_half(idx_v, h, buf)
                pltpu.async_copy(buf, dst, wsem)

            @pl.when(s + 2 < seq)
            def _():
                pltpu.async_copy(
                    xt_hbm.at[s + 2, pl.ds(col0, lk_w)], idx_v, isem
                )

        def body(p, carry):
            phase(2 * p, idx_a, isem_a)
            phase(2 * p + 1, idx_b, isem_b)
            return carry

        lax.fori_loop(0, seq // 2, body, 0)
        pltpu.make_async_copy(
            buf_a, z_hbm.at[seq - 1, :, pl.ds(it_w * w, half)], wsem_a
        ).wait()
        pltpu.make_async_copy(
            buf_b, z_hbm.at[seq - 1, :, pl.ds(it_w * w + half, half)], wsem_b
        ).wait()

    return sc_lookup


def kernel(x, emb_table):
    batch, seq = x.shape
    n_var, d_model = emb_table.shape
    assert d_model == D_MODEL and batch % (LANES * NW * 2) == 0 and seq % 2 == 0
    xt = x.astype(jnp.int32).T
    tab = jnp.pad(emb_table, ((0, 0), (0, 1))).reshape(-1)
    z = _make_sc_lookup(batch, seq, n_var)(xt, tab)
    return z.transpose(2, 4, 0, 1, 3).reshape(batch, seq, D_MODEL)
